# Initial kernel scaffold; baseline (speedup 1.0000x reference)
import jax
import jax.numpy as jnp
from jax.experimental import pallas as pl


def kernel(indices, tables):
    raise NotImplementedError("write your pallas kernel here")


# speedup vs baseline: 1.1420x; 1.1420x over previous
import functools

import jax
import jax.numpy as jnp
from jax import lax
from jax.experimental import pallas as pl
from jax.experimental.pallas import tpu as pltpu
from jax.experimental.pallas import tpu_sc as plsc

_F = 26
_V = 100000
_E = 16
_B = 16384

_NC = 2
_NS = 16
_NW = _NC * _NS
_ROWS = _B * _F
_RPW = _ROWS // _NW
_CHUNK = 26 * 64
_NCHUNK = _RPW // _CHUNK
_NVEC = _CHUNK // 16


def _sc_gather(table_flat, idx_flat, offs):
    mesh = plsc.VectorSubcoreMesh(core_axis_name="c", subcore_axis_name="s")

    @functools.partial(
        pl.kernel,
        mesh=mesh,
        compiler_params=pltpu.CompilerParams(use_tc_tiling_on_sc=False),
        out_type=jax.ShapeDtypeStruct((_ROWS, _E), jnp.float32),
        scratch_types=[
            pltpu.VMEM((_CHUNK,), jnp.int32),
            pltpu.VMEM((_CHUNK,), jnp.int32),
            pltpu.VMEM((_CHUNK, _E), jnp.float32),
            pltpu.SemaphoreType.DMA,
        ],
    )
    def k(table_hbm, idx_hbm, offs_hbm, out_hbm, offs_v, idx_v, rows_v, sem):
        wid = lax.axis_index("s") * _NC + lax.axis_index("c")
        base0 = wid * _RPW
        pltpu.sync_copy(offs_hbm, offs_v)

        def chunk_body(j, carry):
            base = base0 + j * _CHUNK
            pltpu.sync_copy(idx_hbm.at[pl.ds(base, _CHUNK)], idx_v)

            def add_body(i, c):
                s = pl.ds(i * 16, 16)
                idx_v[s] = idx_v[s] + offs_v[s]
                return c

            lax.fori_loop(0, _NVEC, add_body, 0)
            pltpu.async_copy(table_hbm.at[idx_v], rows_v, sem).wait()
            pltpu.sync_copy(rows_v, out_hbm.at[pl.ds(base, _CHUNK)])
            return carry

        lax.fori_loop(0, _NCHUNK, chunk_body, 0)

    return k(table_flat, idx_flat, offs)


def kernel(indices, tables):
    f, v, e = tables.shape
    table_flat = tables.reshape(f * v, e)
    idx_flat = indices.reshape(-1).astype(jnp.int32)
    offs = jnp.tile(jnp.arange(f, dtype=jnp.int32) * v, _CHUNK // f)
    out = _sc_gather(table_flat, idx_flat, offs)
    return out.reshape(_B, f * e)


# speedup vs baseline: 6.4466x; 5.6449x over previous
import functools

import jax
import jax.numpy as jnp
from jax import lax
from jax.experimental import pallas as pl
from jax.experimental.pallas import tpu as pltpu
from jax.experimental.pallas import tpu_sc as plsc

_F = 26
_V = 100000
_E = 16
_B = 16384

_NC = 2
_NS = 16
_NW = _NC * _NS
_R = _F * _E
_RPW = _R // _NW
_HB = _B // 2


def _sc_lookup(tables_t, idx_t):
    mesh = plsc.VectorSubcoreMesh(core_axis_name="c", subcore_axis_name="s")

    @functools.partial(
        pl.kernel,
        mesh=mesh,
        compiler_params=pltpu.CompilerParams(needs_layout_passes=False),
        out_type=jax.ShapeDtypeStruct((_R, _B), jnp.float32),
        scratch_types=[
            pltpu.VMEM((_V,), jnp.float32),
            pltpu.VMEM((_HB,), jnp.int32),
            pltpu.VMEM((_HB,), jnp.float32),
        ],
    )
    def k(tbl_hbm, idx_hbm, out_hbm, row_v, idx_v, out_v):
        wid = lax.axis_index("s") * _NC + lax.axis_index("c")

        def row_body(j, carry):
            r = wid * _RPW + j
            f = r // _E
            pltpu.sync_copy(tbl_hbm.at[r], row_v)

            def half_body(h, c2):
                b0 = h * _HB
                pltpu.sync_copy(idx_hbm.at[f, pl.ds(b0, _HB)], idx_v)

                def g_body(i, c3):
                    s = pl.ds(i * 16, 16)
                    out_v[s] = plsc.load_gather(row_v, [idx_v[s]])
                    return c3

                lax.fori_loop(0, _HB // 16, g_body, 0)
                pltpu.sync_copy(out_v, out_hbm.at[r, pl.ds(b0, _HB)])
                return c2

            lax.fori_loop(0, 2, half_body, 0)
            return carry

        lax.fori_loop(0, _RPW, row_body, 0)

    return k(tables_t, idx_t)


def kernel(indices, tables):
    f, v, e = tables.shape
    tables_t = tables.transpose(0, 2, 1).reshape(f * e, v)
    idx_t = indices.T
    out_t = _sc_lookup(tables_t, idx_t)
    return out_t.reshape(f, e, _B).transpose(2, 0, 1).reshape(_B, f * e)


# speedup vs baseline: 7.5835x; 1.1764x over previous
import functools

import jax
import jax.numpy as jnp
from jax import lax
from jax.experimental import pallas as pl
from jax.experimental.pallas import tpu as pltpu
from jax.experimental.pallas import tpu_sc as plsc

_F = 26
_V = 100000
_E = 16
_B = 16384

_NC = 2
_NS = 16
_NW = _NC * _NS
_R = _F * _E
_RPW = _R // _NW
_HB = _B // 2
_UNROLL = 8


def _sc_lookup(tables_t, idx_t):
    mesh = plsc.VectorSubcoreMesh(core_axis_name="c", subcore_axis_name="s")

    @functools.partial(
        pl.kernel,
        mesh=mesh,
        compiler_params=pltpu.CompilerParams(needs_layout_passes=False),
        out_type=jax.ShapeDtypeStruct((_R, _B), jnp.float32),
        scratch_types=[
            pltpu.VMEM((_V,), jnp.float32),
            pltpu.VMEM((_HB,), jnp.int32),
            pltpu.VMEM((_HB,), jnp.float32),
            pltpu.VMEM((_HB,), jnp.float32),
            pltpu.SemaphoreType.DMA,
        ],
    )
    def k(tbl_hbm, idx_hbm, out_hbm, row_v, idx_v, out0_v, out1_v, wsem):
        wid = lax.axis_index("s") * _NC + lax.axis_index("c")

        def row_body(j, carry):
            r = wid * _RPW + j
            f = r // _E
            pltpu.sync_copy(tbl_hbm.at[r], row_v)

            def half_body(h, c2):
                b0 = h * _HB
                out_v = out0_v if h == 0 else out1_v
                pltpu.sync_copy(idx_hbm.at[f, pl.ds(b0, _HB)], idx_v)

                def g_body(i, c3):
                    base = i * (16 * _UNROLL)
                    for u in range(_UNROLL):
                        s = pl.ds(base + u * 16, 16)
                        out_v[s] = plsc.load_gather(row_v, [idx_v[s]])
                    return c3

                lax.fori_loop(0, _HB // (16 * _UNROLL), g_body, 0)
                pltpu.async_copy(out_v, out_hbm.at[r, pl.ds(b0, _HB)], wsem)
                return c2

            half_body(0, 0)
            half_body(1, 0)
            pltpu.make_async_copy(
                out0_v, out_hbm.at[r, pl.ds(0, _HB)], wsem).wait()
            pltpu.make_async_copy(
                out1_v, out_hbm.at[r, pl.ds(_HB, _HB)], wsem).wait()
            return carry

        lax.fori_loop(0, _RPW, row_body, 0)

    return k(tables_t, idx_t)


def kernel(indices, tables):
    f, v, e = tables.shape
    tables_t = tables.transpose(0, 2, 1).reshape(f * e, v)
    idx_t = indices.T
    out_t = _sc_lookup(tables_t, idx_t)
    return out_t.reshape(f, e, _B).transpose(2, 0, 1).reshape(_B, f * e)


# speedup vs baseline: 9.1135x; 1.2017x over previous
import functools

import jax
import jax.numpy as jnp
from jax import lax
from jax.experimental import pallas as pl
from jax.experimental.pallas import tpu as pltpu
from jax.experimental.pallas import tpu_sc as plsc

_F = 26
_V = 100000
_E = 16
_B = 16384

_NC = 2
_NS = 16
_NW = _NC * _NS
_R = _F * _E
_RPW = _R // _NW
_NQ = 4
_QB = _B // _NQ
_UNROLL = 8


def _sc_lookup(tables_t, idx_t):
    mesh = plsc.VectorSubcoreMesh(core_axis_name="c", subcore_axis_name="s")

    @functools.partial(
        pl.kernel,
        mesh=mesh,
        compiler_params=pltpu.CompilerParams(needs_layout_passes=False),
        out_type=jax.ShapeDtypeStruct((_R, _B), jnp.float32),
        scratch_types=[
            pltpu.VMEM((_V,), jnp.float32),
            pltpu.VMEM((_QB,), jnp.int32),
            pltpu.VMEM((_QB,), jnp.int32),
            pltpu.VMEM((_QB,), jnp.float32),
            pltpu.VMEM((_QB,), jnp.float32),
            pltpu.SemaphoreType.DMA,
            pltpu.SemaphoreType.DMA,
        ],
    )
    def k(tbl_hbm, idx_hbm, out_hbm, row_v, ib0, ib1, ob0, ob1, isem, wsem):
        wid = lax.axis_index("s") * _NC + lax.axis_index("c")
        ibufs = (ib0, ib1)
        obufs = (ob0, ob1)

        r0 = wid * _RPW
        pltpu.async_copy(idx_hbm.at[r0 // _E, pl.ds(0, _QB)], ibufs[0], isem)

        def row_body(j, carry):
            r = wid * _RPW + j
            f = r // _E
            pltpu.sync_copy(tbl_hbm.at[r], row_v)

            for q in range(_NQ):
                b0 = q * _QB
                ib = ibufs[q % 2]
                ob = obufs[q % 2]

                pltpu.make_async_copy(
                    idx_hbm.at[f, pl.ds(b0, _QB)], ib, isem).wait()

                if q < _NQ - 1:
                    pltpu.async_copy(
                        idx_hbm.at[f, pl.ds(b0 + _QB, _QB)],
                        ibufs[(q + 1) % 2], isem)
                else:
                    @pl.when(j < _RPW - 1)
                    def _():
                        fn = (r + 1) // _E
                        pltpu.async_copy(
                            idx_hbm.at[fn, pl.ds(0, _QB)],
                            ibufs[(q + 1) % 2], isem)

                if q >= 2:
                    pltpu.make_async_copy(
                        ob, out_hbm.at[r, pl.ds(b0 - 2 * _QB, _QB)],
                        wsem).wait()
                else:
                    @pl.when(j > 0)
                    def _():
                        pltpu.make_async_copy(
                            ob, out_hbm.at[r - 1, pl.ds(b0 + 2 * _QB, _QB)],
                            wsem).wait()

                def g_body(i, c3):
                    base = i * (16 * _UNROLL)
                    for u in range(_UNROLL):
                        s = pl.ds(base + u * 16, 16)
                        ob[s] = plsc.load_gather(row_v, [ib[s]])
                    return c3

                lax.fori_loop(0, _QB // (16 * _UNROLL), g_body, 0)
                pltpu.async_copy(ob, out_hbm.at[r, pl.ds(b0, _QB)], wsem)

            return carry

        lax.fori_loop(0, _RPW, row_body, 0)

        rl = wid * _RPW + _RPW - 1
        pltpu.make_async_copy(
            obufs[0], out_hbm.at[rl, pl.ds(2 * _QB, _QB)], wsem).wait()
        pltpu.make_async_copy(
            obufs[1], out_hbm.at[rl, pl.ds(3 * _QB, _QB)], wsem).wait()

    return k(tables_t, idx_t)


def kernel(indices, tables):
    f, v, e = tables.shape
    tables_t = tables.transpose(0, 2, 1).reshape(f * e, v)
    idx_t = indices.T
    out_t = _sc_lookup(tables_t, idx_t)
    return out_t.reshape(f, e, _B).transpose(2, 0, 1).reshape(_B, f * e)


# speedup vs baseline: 9.1446x; 1.0034x over previous
import functools

import jax
import jax.numpy as jnp
from jax import lax
from jax.experimental import pallas as pl
from jax.experimental.pallas import tpu as pltpu
from jax.experimental.pallas import tpu_sc as plsc

_F = 26
_V = 100000
_E = 16
_B = 16384

_NC = 2
_NS = 16
_NW = _NC * _NS
_R = _F * _E
_RPW = _R // _NW
_NQ = 4
_QB = _B // _NQ
_UNROLL = 16


def _sc_lookup(tables_t, idx_t):
    mesh = plsc.VectorSubcoreMesh(core_axis_name="c", subcore_axis_name="s")

    @functools.partial(
        pl.kernel,
        mesh=mesh,
        compiler_params=pltpu.CompilerParams(needs_layout_passes=False),
        out_type=jax.ShapeDtypeStruct((_R, _B), jnp.float32),
        scratch_types=[
            pltpu.VMEM((_V,), jnp.float32),
            pltpu.VMEM((_QB,), jnp.int32),
            pltpu.VMEM((_QB,), jnp.int32),
            pltpu.VMEM((_QB,), jnp.float32),
            pltpu.VMEM((_QB,), jnp.float32),
            pltpu.SemaphoreType.DMA,
            pltpu.SemaphoreType.DMA,
        ],
    )
    def k(tbl_hbm, idx_hbm, out_hbm, row_v, ib0, ib1, ob0, ob1, isem, wsem):
        wid = lax.axis_index("s") * _NC + lax.axis_index("c")
        ibufs = (ib0, ib1)
        obufs = (ob0, ob1)

        r0 = wid * _RPW
        pltpu.async_copy(idx_hbm.at[r0 // _E, pl.ds(0, _QB)], ibufs[0], isem)

        def row_body(j, carry):
            r = wid * _RPW + j
            f = r // _E
            pltpu.sync_copy(tbl_hbm.at[r], row_v)

            for q in range(_NQ):
                b0 = q * _QB
                ib = ibufs[q % 2]
                ob = obufs[q % 2]

                pltpu.make_async_copy(
                    idx_hbm.at[f, pl.ds(b0, _QB)], ib, isem).wait()

                if q < _NQ - 1:
                    pltpu.async_copy(
                        idx_hbm.at[f, pl.ds(b0 + _QB, _QB)],
                        ibufs[(q + 1) % 2], isem)
                else:
                    @pl.when(j < _RPW - 1)
                    def _():
                        fn = (r + 1) // _E
                        pltpu.async_copy(
                            idx_hbm.at[fn, pl.ds(0, _QB)],
                            ibufs[(q + 1) % 2], isem)

                if q >= 2:
                    pltpu.make_async_copy(
                        ob, out_hbm.at[r, pl.ds(b0 - 2 * _QB, _QB)],
                        wsem).wait()
                else:
                    @pl.when(j > 0)
                    def _():
                        pltpu.make_async_copy(
                            ob, out_hbm.at[r - 1, pl.ds(b0 + 2 * _QB, _QB)],
                            wsem).wait()

                def g_body(i, c3):
                    base = i * (16 * _UNROLL)
                    for u in range(_UNROLL):
                        s = pl.ds(base + u * 16, 16)
                        ob[s] = plsc.load_gather(row_v, [ib[s]])
                    return c3

                lax.fori_loop(0, _QB // (16 * _UNROLL), g_body, 0)
                pltpu.async_copy(ob, out_hbm.at[r, pl.ds(b0, _QB)], wsem)

            return carry

        lax.fori_loop(0, _RPW, row_body, 0)

        rl = wid * _RPW + _RPW - 1
        pltpu.make_async_copy(
            obufs[0], out_hbm.at[rl, pl.ds(2 * _QB, _QB)], wsem).wait()
        pltpu.make_async_copy(
            obufs[1], out_hbm.at[rl, pl.ds(3 * _QB, _QB)], wsem).wait()

    return k(tables_t, idx_t)


def kernel(indices, tables):
    f, v, e = tables.shape
    tables_t = tables.transpose(0, 2, 1).reshape(f * e, v)
    idx_t = indices.T
    out_t = _sc_lookup(tables_t, idx_t)
    return out_t.reshape(f, e, _B).transpose(2, 0, 1).reshape(_B, f * e)
